# baseline (speedup 1.0000x reference)
import functools

import jax
import jax.numpy as jnp
from jax import lax
from jax.experimental import pallas as pl
from jax.experimental.pallas import tpu as pltpu
from jax.experimental.pallas import tpu_sc as plsc

B = 16384
L = 20
ED = 32
HID = 128
OUT = 64

NC = 2
NS = 16
NW = NC * NS
BPW = B // NW
ACH = 128
NACH = BPW // ACH

_MESH = plsc.VectorSubcoreMesh(core_axis_name="c", subcore_axis_name="s")
_SC_PARAMS = pltpu.CompilerParams(needs_layout_passes=False,
                                  use_tc_tiling_on_sc=False)


def _pool_body(tt_idx, ps_idx, lt_idx, tg_idx, at_idx,
               tt_flat, ps_flat, lt_pack, tg_pack, at_pack,
               feat_out,
               tabf, tabp, idxs, idxf, outf, sem_i):
    wid = lax.axis_index("s") * NC + lax.axis_index("c")
    base = wid * BPW
    iota16 = lax.iota(jnp.int32, 16)

    idx_descs = [
        pltpu.async_copy(h.at[pl.ds(base * L, BPW * L)],
                         idxf.at[pl.ds(i * BPW * L, BPW * L)], sem_i)
        for i, h in enumerate((lt_idx, tg_idx, at_idx))
    ]

    def pool(tab_hbm, nrows, idx_load, lf, fcol, scale, packed):
        if packed:
            pltpu.sync_copy(tab_hbm, tabp)
        else:
            pltpu.sync_copy(tab_hbm, tabf.at[pl.ds(0, nrows * ED)])

        def gbody(g, carry):
            g16 = g * 16

            def lbody(l, accs):
                iv = idx_load(g16, l)
                if not packed:
                    return tuple(
                        accs[d] + plsc.load_gather(tabf, [iv + d * nrows])
                        for d in range(ED)
                    )
                new = list(accs)
                for dp in range(ED // 2):
                    w = plsc.load_gather(tabp, [iv + dp * nrows])
                    lo = plsc.bitcast(w << 16, jnp.float32)
                    hi = plsc.bitcast(w & jnp.int32(-65536), jnp.float32)
                    new[2 * dp] = new[2 * dp] + lo
                    new[2 * dp + 1] = new[2 * dp + 1] + hi
                return tuple(new)

            accs = tuple(jnp.zeros((16,), jnp.float32) for _ in range(ED))
            accs = lax.fori_loop(0, lf, lbody, accs)
            rowv = g16 + iota16
            r_hi = rowv >> 3
            r_lo = rowv & 7
            for d in range(ED):
                v = accs[d] * scale if scale != 1.0 else accs[d]
                plsc.store_scatter(
                    outf, [r_hi, r_lo, jnp.full((16,), d, jnp.int32)], v)
            return carry

        lax.fori_loop(0, BPW // 16, gbody, 0)
        tc, off = fcol
        pltpu.sync_copy(outf.at[:, :, pl.ds(0, ED)],
                        feat_out.at[pl.ds(wid * (BPW // 8), BPW // 8),
                                    tc, slice(None), pl.ds(off, ED)])

    def small_idx(idx_hbm):
        pltpu.sync_copy(idx_hbm.at[pl.ds(base, BPW)], idxs)

        def load(g16, l):
            return idxs[pl.ds(g16, 16)]
        return load

    def big_idx(i, desc):
        desc.wait()

        def load(g16, l):
            rb = (g16 + iota16) * L
            return plsc.load_gather(idxf, [rb + (i * BPW * L + l)])
        return load

    pool(tt_flat, 7, small_idx(tt_idx), 1, (0, 0), 1.0, False)
    pool(ps_flat, 2, small_idx(ps_idx), 1, (0, ED), 1.0, False)
    pool(lt_pack, 1000, big_idx(0, idx_descs[0]), L, (0, 2 * ED),
         1.0 / L, True)
    pool(tg_pack, 1000, big_idx(1, idx_descs[1]), L, (1, 0),
         1.0 / L, True)
    pool(at_pack, 1000, big_idx(2, idx_descs[2]), L, (1, ED),
         1.0 / L, True)


def _addr_body(addr_idx2, addr_tab, addr_out, iv2, rows_v, sem_a):
    wid = lax.axis_index("s") * NC + lax.axis_index("c")
    base = wid * BPW
    pltpu.sync_copy(addr_idx2.at[pl.ds(wid * NACH, NACH)], iv2)
    descs = [
        pltpu.async_copy(addr_tab.at[iv2.at[j]],
                         rows_v.at[pl.ds(j * ACH, ACH)], sem_a)
        for j in range(NACH)
    ]
    for d in descs:
        d.wait()
    pltpu.sync_copy(rows_v, addr_out.at[pl.ds(base, BPW)])


def _pack_bf16_T(tab):
    t = tab.astype(jnp.bfloat16).reshape(-1, ED // 2, 2)
    w = jax.lax.bitcast_convert_type(t, jnp.int32)
    return w.T.reshape(-1)


@jax.jit
def _embed(address, tourist_type, price_sensitive, lt_idx, tg_idx, at_idx,
           addr_table, tt_table, ps_table, lt_table, tg_table, at_table):
    f32 = jnp.float32
    pool_fn = pl.kernel(
        _pool_body,
        out_type=[jax.ShapeDtypeStruct((B // 8, 2, 8, 128), f32)],
        mesh=_MESH,
        scratch_types=[
            pltpu.VMEM((7 * ED,), f32),
            pltpu.VMEM((1000 * ED // 2,), jnp.int32),
            pltpu.VMEM((BPW,), jnp.int32),
            pltpu.VMEM((3 * L * BPW,), jnp.int32),
            pltpu.VMEM((BPW // 8, 8, ED + 1), f32),
            pltpu.SemaphoreType.DMA,
        ],
        compiler_params=_SC_PARAMS)
    addr_fn = pl.kernel(
        _addr_body,
        out_type=[jax.ShapeDtypeStruct((B, ED), f32)],
        mesh=_MESH,
        scratch_types=[
            pltpu.VMEM((NACH, ACH), jnp.int32),
            pltpu.VMEM((BPW, ED), f32),
            pltpu.SemaphoreType.DMA,
        ],
        compiler_params=_SC_PARAMS)
    (feat,) = pool_fn(tourist_type, price_sensitive,
                      lt_idx.reshape(-1), tg_idx.reshape(-1),
                      at_idx.reshape(-1),
                      tt_table.T.reshape(-1), ps_table.T.reshape(-1),
                      _pack_bf16_T(lt_table), _pack_bf16_T(tg_table),
                      _pack_bf16_T(at_table))
    (addr_rows,) = addr_fn(address.reshape(B // ACH, ACH), addr_table)
    return feat, addr_rows


def _mlp_body(x, a, w1, b1, w2, b2, o):
    x4 = x[...]
    bm = x4.shape[0] * 8
    xa = x4[:, 0, :, 0:96].reshape(bm, 96)
    xb = x4[:, 1, :, 0:64].reshape(bm, 64)
    f32 = jnp.float32
    h = jnp.dot(a[...], w1[0:ED], preferred_element_type=f32)
    h = h + jnp.dot(xa, w1[ED:ED + 96], preferred_element_type=f32)
    h = h + jnp.dot(xb, w1[ED + 96:], preferred_element_type=f32)
    h = jnp.maximum(h + b1[...], 0.0)
    y = jnp.dot(h, w2[...], preferred_element_type=f32) + b2[...]
    ss = jnp.sum(y * y, axis=1, keepdims=True)
    n = jnp.maximum(jnp.sqrt(ss), 1e-12)
    o[...] = y / n


@functools.partial(jax.jit, static_argnames=("bm",))
def _mlp(x4, a, W1, b1, W2, b2, bm=2048):
    grid = (B // bm,)
    z = lambda i: (0, 0)
    return pl.pallas_call(
        _mlp_body,
        grid=grid,
        in_specs=[
            pl.BlockSpec((bm // 8, 2, 8, 128), lambda i: (i, 0, 0, 0)),
            pl.BlockSpec((bm, ED), lambda i: (i, 0)),
            pl.BlockSpec((6 * ED, HID), z), pl.BlockSpec((1, HID), z),
            pl.BlockSpec((HID, OUT), z), pl.BlockSpec((1, OUT), z),
        ],
        out_specs=pl.BlockSpec((bm, OUT), lambda i: (i, 0)),
        out_shape=jax.ShapeDtypeStruct((B, OUT), jnp.float32),
    )(x4, a, W1, b1, W2, b2)


def kernel(address, tourist_type, price_sensitive, like_type, targets,
           attention, addr_table, tt_table, ps_table, lt_table, tg_table,
           at_table, W1, b1, W2, b2):
    i32 = jnp.int32
    feat, addr_rows = _embed(
        address.astype(i32), tourist_type.astype(i32),
        price_sensitive.astype(i32), like_type.astype(i32),
        targets.astype(i32), attention.astype(i32),
        addr_table, tt_table, ps_table, lt_table, tg_table, at_table)
    return _mlp(feat, addr_rows, W1, b1.reshape(1, HID), W2,
                b2.reshape(1, OUT))


# speedup vs baseline: 1.1260x; 1.1260x over previous
import functools

import jax
import jax.numpy as jnp
from jax import lax
from jax.experimental import pallas as pl
from jax.experimental.pallas import tpu as pltpu
from jax.experimental.pallas import tpu_sc as plsc

B = 16384
L = 20
LP = 24
ED = 32
HID = 128
OUT = 64

NC = 2
NS = 16
NW = NC * NS
BPW = B // NW
ACH = 128
NACH = BPW // ACH

_MESH = plsc.VectorSubcoreMesh(core_axis_name="c", subcore_axis_name="s")


def _embed_body(addr_idx2, tt_idx, ps_idx, idxT,
                addr_tab, tt_flat, ps_flat, lt_pack, tg_pack, at_pack,
                feat_out,
                iv2, rows_v, tabf, tabp, idxs, idxf, outf, sem_a, sem_i):
    wid = lax.axis_index("s") * NC + lax.axis_index("c")
    base = wid * BPW

    idx_descs = [
        pltpu.async_copy(idxT.at[t * LP + l, pl.ds(base, BPW)],
                         idxf.at[pl.ds((t * L + l) * BPW, BPW)], sem_i)
        for t in range(3) for l in range(L)
    ]

    pltpu.sync_copy(addr_idx2.at[pl.ds(wid * NACH, NACH)], iv2)
    addr_descs = [
        pltpu.async_copy(addr_tab.at[iv2.at[j]],
                         rows_v.at[pl.ds(j * ACH, ACH)], sem_a)
        for j in range(NACH)
    ]

    iota16 = lax.iota(jnp.int32, 16)

    def pool(tab_hbm, nrows, idx_load, lf, fcol, scale, packed):
        if packed:
            pltpu.sync_copy(tab_hbm, tabp)
        else:
            pltpu.sync_copy(tab_hbm, tabf.at[pl.ds(0, nrows * ED)])

        def gbody(g, carry):
            g16 = g * 16

            def lbody(l, accs):
                iv = idx_load(g16, l)
                if not packed:
                    return tuple(
                        accs[d] + plsc.load_gather(tabf, [iv + d * nrows])
                        for d in range(ED)
                    )
                new = list(accs)
                for dp in range(ED // 2):
                    w = plsc.load_gather(tabp, [iv + dp * nrows])
                    lo = plsc.bitcast(w << 16, jnp.float32)
                    hi = plsc.bitcast(w & jnp.int32(-65536), jnp.float32)
                    new[2 * dp] = new[2 * dp] + lo
                    new[2 * dp + 1] = new[2 * dp + 1] + hi
                return tuple(new)

            accs = tuple(jnp.zeros((16,), jnp.float32) for _ in range(ED))
            accs = lax.fori_loop(0, lf, lbody, accs)
            rowv = g16 + iota16
            r_hi = rowv >> 3
            r_lo = rowv & 7
            for d in range(ED):
                v = accs[d] * scale if scale != 1.0 else accs[d]
                plsc.store_scatter(
                    outf, [r_hi, r_lo, jnp.full((16,), d, jnp.int32)], v)
            return carry

        lax.fori_loop(0, BPW // 16, gbody, 0)
        tc, off = divmod(fcol, 128)
        pltpu.sync_copy(outf.at[:, :, pl.ds(0, ED)],
                        feat_out.at[pl.ds(wid * (BPW // 8), BPW // 8),
                                    tc, slice(None), pl.ds(off, ED)])

    def small_idx(idx_hbm):
        pltpu.sync_copy(idx_hbm.at[pl.ds(base, BPW)], idxs)

        def load(g16, l):
            return idxs[pl.ds(g16, 16)]
        return load

    def big_idx(t):
        def load(g16, l):
            return idxf[pl.ds((t * L + l) * BPW + g16, 16)]
        return load

    pool(tt_flat, 7, small_idx(tt_idx), 1, ED, 1.0, False)
    pool(ps_flat, 2, small_idx(ps_idx), 1, 2 * ED, 1.0, False)
    for d in idx_descs:
        d.wait()
    pool(lt_pack, 1000, big_idx(0), L, 3 * ED, 1.0 / L, True)
    pool(tg_pack, 1000, big_idx(1), L, 4 * ED, 1.0 / L, True)
    pool(at_pack, 1000, big_idx(2), L, 5 * ED, 1.0 / L, True)

    for d in addr_descs:
        d.wait()
    out_descs = [
        pltpu.async_copy(rows_v.at[pl.ds(r8 * 8, 8)],
                         feat_out.at[wid * (BPW // 8) + r8, 0,
                                     slice(None), pl.ds(0, ED)], sem_a)
        for r8 in range(BPW // 8)
    ]
    for d in out_descs:
        d.wait()


def _pack_bf16_T(tab):
    t = tab.astype(jnp.bfloat16).reshape(-1, ED // 2, 2)
    w = jax.lax.bitcast_convert_type(t, jnp.int32)
    return w.T.reshape(-1)


def _idxT_body(lt, tg, at, o):
    z = jnp.zeros((LP - L, lt.shape[0]), jnp.int32)
    parts = []
    for x in (lt, tg, at):
        parts.append(jnp.transpose(x[...], (1, 0)))
        parts.append(z)
    o[...] = jnp.concatenate(parts, axis=0)


@functools.partial(jax.jit, static_argnames=("bm",))
def _idxT(lt, tg, at, bm=2048):
    return pl.pallas_call(
        _idxT_body,
        grid=(B // bm,),
        in_specs=[pl.BlockSpec((bm, L), lambda i: (i, 0))] * 3,
        out_specs=pl.BlockSpec((3 * LP, bm), lambda i: (0, i)),
        out_shape=jax.ShapeDtypeStruct((3 * LP, B), jnp.int32),
    )(lt, tg, at)


@jax.jit
def _embed(address, tourist_type, price_sensitive, idxT,
           addr_table, tt_table, ps_table, lt_table, tg_table, at_table):
    f32 = jnp.float32
    out_type = [
        jax.ShapeDtypeStruct((B // 8, 2, 8, 128), f32),
    ]
    scratch = [
        pltpu.VMEM((NACH, ACH), jnp.int32),
        pltpu.VMEM((BPW, ED), f32),
        pltpu.VMEM((7 * ED,), f32),
        pltpu.VMEM((1000 * ED // 2,), jnp.int32),
        pltpu.VMEM((BPW,), jnp.int32),
        pltpu.VMEM((3 * L * BPW,), jnp.int32),
        pltpu.VMEM((BPW // 8, 8, ED + 1), f32),
        pltpu.SemaphoreType.DMA,
        pltpu.SemaphoreType.DMA,
    ]
    fn = pl.kernel(_embed_body, out_type=out_type, mesh=_MESH,
                   scratch_types=scratch,
                   compiler_params=pltpu.CompilerParams(
                       needs_layout_passes=False,
                       use_tc_tiling_on_sc=False))
    (feat,) = fn(address.reshape(B // ACH, ACH), tourist_type,
                 price_sensitive, idxT, addr_table,
                 tt_table.T.reshape(-1), ps_table.T.reshape(-1),
                 _pack_bf16_T(lt_table), _pack_bf16_T(tg_table),
                 _pack_bf16_T(at_table))
    return feat


def _mlp_body(x, w1, b1, w2, b2, o):
    x4 = x[...]
    bm = x4.shape[0] * 8
    xa = x4[:, 0].reshape(bm, 128)
    xb = x4[:, 1, :, 0:64].reshape(bm, 64)
    xx = jnp.concatenate([xa, xb], axis=1)
    h = jnp.dot(xx, w1[...], preferred_element_type=jnp.float32) + b1[...]
    h = jnp.maximum(h, 0.0)
    y = jnp.dot(h, w2[...], preferred_element_type=jnp.float32) + b2[...]
    ss = jnp.sum(y * y, axis=1, keepdims=True)
    n = jnp.maximum(jnp.sqrt(ss), 1e-12)
    o[...] = y / n


@functools.partial(jax.jit, static_argnames=("bm",))
def _mlp(x4, W1, b1, W2, b2, bm=2048):
    grid = (B // bm,)
    z = lambda i: (0, 0)
    return pl.pallas_call(
        _mlp_body,
        grid=grid,
        in_specs=[
            pl.BlockSpec((bm // 8, 2, 8, 128), lambda i: (i, 0, 0, 0)),
            pl.BlockSpec((6 * ED, HID), z), pl.BlockSpec((1, HID), z),
            pl.BlockSpec((HID, OUT), z), pl.BlockSpec((1, OUT), z),
        ],
        out_specs=pl.BlockSpec((bm, OUT), lambda i: (i, 0)),
        out_shape=jax.ShapeDtypeStruct((B, OUT), jnp.float32),
    )(x4, W1, b1, W2, b2)


def kernel(address, tourist_type, price_sensitive, like_type, targets,
           attention, addr_table, tt_table, ps_table, lt_table, tg_table,
           at_table, W1, b1, W2, b2):
    i32 = jnp.int32
    idxT = _idxT(like_type.astype(i32), targets.astype(i32),
                 attention.astype(i32))
    feat = _embed(
        address.astype(i32), tourist_type.astype(i32),
        price_sensitive.astype(i32), idxT,
        addr_table, tt_table, ps_table, lt_table, tg_table, at_table)
    return _mlp(feat, W1, b1.reshape(1, HID), W2, b2.reshape(1, OUT))


# speedup vs baseline: 1.2090x; 1.0737x over previous
import functools

import jax
import jax.numpy as jnp
from jax import lax
from jax.experimental import pallas as pl
from jax.experimental.pallas import tpu as pltpu
from jax.experimental.pallas import tpu_sc as plsc

B = 16384
L = 20
ED = 32
NF = 6
FD = NF * ED
HID = 128
OUT = 64

NC = 2
NS = 16
NW = NC * NS
BPW = B // NW
ACH = 128
NACH = BPW // ACH

_MESH = plsc.VectorSubcoreMesh(core_axis_name="c", subcore_axis_name="s")


def _embed_body(addr_idx2, tt_idx, ps_idx, idx3,
                addr_tab, tt_flat, ps_flat, lt_pack, tg_pack, at_pack,
                feat_out,
                iv2, rows_v, tabf, tabp, idxs, idxf, outf, sem_a, sem_i):
    wid = lax.axis_index("s") * NC + lax.axis_index("c")
    base = wid * BPW

    d_idx3 = pltpu.async_copy(idx3.at[pl.ds(base * 3 * L, BPW * 3 * L)],
                              idxf, sem_i)

    pltpu.sync_copy(addr_idx2.at[pl.ds(wid * NACH, NACH)], iv2)
    addr_descs = [
        pltpu.async_copy(addr_tab.at[iv2.at[j]],
                         rows_v.at[pl.ds(j * ACH, ACH)], sem_a)
        for j in range(NACH)
    ]

    iota16 = lax.iota(jnp.int32, 16)

    def pool(tab_hbm, nrows, idx_load, lf, fcol, scale, packed):
        if packed:
            pltpu.sync_copy(tab_hbm, tabp)
        else:
            pltpu.sync_copy(tab_hbm, tabf.at[pl.ds(0, nrows * ED)])

        def gbody(g, carry):
            g16 = g * 16

            def lbody(l, accs):
                iv = idx_load(g16, l)
                if not packed:
                    return tuple(
                        accs[d] + plsc.load_gather(tabf, [iv + d * nrows])
                        for d in range(ED)
                    )
                new = list(accs)
                for dp in range(ED // 2):
                    w = plsc.load_gather(tabp, [iv + dp * nrows])
                    lo = plsc.bitcast(w << 16, jnp.float32)
                    hi = plsc.bitcast(w & jnp.int32(-65536), jnp.float32)
                    new[2 * dp] = new[2 * dp] + lo
                    new[2 * dp + 1] = new[2 * dp + 1] + hi
                return tuple(new)

            accs = tuple(jnp.zeros((16,), jnp.float32) for _ in range(ED))
            accs = lax.fori_loop(0, lf, lbody, accs)
            rowv = g16 + iota16
            r_hi = rowv >> 3
            r_lo = rowv & 7
            for d in range(ED):
                v = accs[d] * scale if scale != 1.0 else accs[d]
                plsc.store_scatter(
                    outf, [r_hi, r_lo, jnp.full((16,), d, jnp.int32)], v)
            return carry

        lax.fori_loop(0, BPW // 16, gbody, 0)
        tc, off = divmod(fcol, 128)
        pltpu.sync_copy(outf.at[:, :, pl.ds(0, ED)],
                        feat_out.at[pl.ds(wid * (BPW // 8), BPW // 8),
                                    tc, slice(None), pl.ds(off, ED)])

    def small_idx(idx_hbm):
        pltpu.sync_copy(idx_hbm.at[pl.ds(base, BPW)], idxs)

        def load(g16, l):
            return idxs[pl.ds(g16, 16)]
        return load

    def fused_idx(toff):
        def load(g16, l):
            rb = (g16 + iota16) * (3 * L)
            return plsc.load_gather(idxf, [rb + (toff + l)])
        return load

    pool(tt_flat, 7, small_idx(tt_idx), 1, ED, 1.0, False)
    pool(ps_flat, 2, small_idx(ps_idx), 1, 2 * ED, 1.0, False)
    d_idx3.wait()
    pool(lt_pack, 1000, fused_idx(0), L, 3 * ED, 1.0 / L, True)
    pool(tg_pack, 1000, fused_idx(L), L, 4 * ED, 1.0 / L, True)
    pool(at_pack, 1000, fused_idx(2 * L), L, 5 * ED, 1.0 / L, True)

    for d in addr_descs:
        d.wait()
    out_descs = [
        pltpu.async_copy(rows_v.at[pl.ds(r8 * 8, 8)],
                         feat_out.at[wid * (BPW // 8) + r8, 0,
                                     slice(None), pl.ds(0, ED)], sem_a)
        for r8 in range(BPW // 8)
    ]
    for d in out_descs:
        d.wait()


def _pack_bf16_T(tab):
    t = tab.astype(jnp.bfloat16).reshape(-1, ED // 2, 2)
    w = jax.lax.bitcast_convert_type(t, jnp.int32)
    return w.T.reshape(-1)


@jax.jit
def _embed(address, tourist_type, price_sensitive, idx3,
           addr_table, tt_table, ps_table, lt_table, tg_table, at_table):
    f32 = jnp.float32
    out_type = [
        jax.ShapeDtypeStruct((B // 8, 2, 8, 128), f32),
    ]
    scratch = [
        pltpu.VMEM((NACH, ACH), jnp.int32),
        pltpu.VMEM((BPW, ED), f32),
        pltpu.VMEM((7 * ED,), f32),
        pltpu.VMEM((1000 * ED // 2,), jnp.int32),
        pltpu.VMEM((BPW,), jnp.int32),
        pltpu.VMEM((3 * L * BPW,), jnp.int32),
        pltpu.VMEM((BPW // 8, 8, ED + 1), f32),
        pltpu.SemaphoreType.DMA,
        pltpu.SemaphoreType.DMA,
    ]
    fn = pl.kernel(_embed_body, out_type=out_type, mesh=_MESH,
                   scratch_types=scratch,
                   compiler_params=pltpu.CompilerParams(
                       needs_layout_passes=False,
                       use_tc_tiling_on_sc=False))
    (feat,) = fn(address.reshape(B // ACH, ACH), tourist_type,
                 price_sensitive, idx3, addr_table,
                 tt_table.T.reshape(-1), ps_table.T.reshape(-1),
                 _pack_bf16_T(lt_table), _pack_bf16_T(tg_table),
                 _pack_bf16_T(at_table))
    return feat


def _mlp_body(x, w1, b1, w2, b2, o):
    x4 = x[...]
    bm = x4.shape[0] * 8
    xa = x4[:, 0].reshape(bm, 128)
    xb = x4[:, 1, :, 0:64].reshape(bm, 64)
    xx = jnp.concatenate([xa, xb], axis=1)
    h = jnp.dot(xx, w1[...], preferred_element_type=jnp.float32) + b1[...]
    h = jnp.maximum(h, 0.0)
    y = jnp.dot(h, w2[...], preferred_element_type=jnp.float32) + b2[...]
    ss = jnp.sum(y * y, axis=1, keepdims=True)
    n = jnp.maximum(jnp.sqrt(ss), 1e-12)
    o[...] = y / n


@functools.partial(jax.jit, static_argnames=("bm",))
def _mlp(x4, W1, b1, W2, b2, bm=2048):
    grid = (B // bm,)
    z = lambda i: (0, 0)
    return pl.pallas_call(
        _mlp_body,
        grid=grid,
        in_specs=[
            pl.BlockSpec((bm // 8, 2, 8, 128), lambda i: (i, 0, 0, 0)),
            pl.BlockSpec((FD, HID), z), pl.BlockSpec((1, HID), z),
            pl.BlockSpec((HID, OUT), z), pl.BlockSpec((1, OUT), z),
        ],
        out_specs=pl.BlockSpec((bm, OUT), lambda i: (i, 0)),
        out_shape=jax.ShapeDtypeStruct((B, OUT), jnp.float32),
    )(x4, W1, b1, W2, b2)


def kernel(address, tourist_type, price_sensitive, like_type, targets,
           attention, addr_table, tt_table, ps_table, lt_table, tg_table,
           at_table, W1, b1, W2, b2):
    i32 = jnp.int32
    idx3 = jnp.concatenate(
        [like_type.astype(i32), targets.astype(i32), attention.astype(i32)],
        axis=1).reshape(-1)
    feat = _embed(
        address.astype(i32), tourist_type.astype(i32),
        price_sensitive.astype(i32), idx3,
        addr_table, tt_table, ps_table, lt_table, tg_table, at_table)
    return _mlp(feat, W1, b1.reshape(1, HID), W2, b2.reshape(1, OUT))
